# baseline (speedup 1.0000x reference)
import jax
import jax.numpy as jnp
from jax.experimental import pallas as pl
from jax.experimental.pallas import tpu as pltpu

D_MODEL = 1024
NUM_EXPERTS = 64
TOPK = 8
N_GROUPS = 8
TOPK_GROUPS = 4
GROUP_SIZE = NUM_EXPERTS // N_GROUPS
N_CAND = TOPK_GROUPS * GROUP_SIZE

TILE = 4096
SPLIT = 1
SUB = TILE // SPLIT


def _gate_half(x, w, wout_ref, iout_ref, h):
    scores = jax.lax.dot_general(
        w, x, (((1,), (1,)), ((), ())), preferred_element_type=jnp.float32
    )
    denom = jnp.sum(jnp.exp(scores), axis=0, keepdims=True)

    giota = jax.lax.broadcasted_iota(jnp.int32, (N_GROUPS, SUB), 0)

    gmax = jnp.concatenate(
        [
            jnp.max(scores[g * GROUP_SIZE : (g + 1) * GROUP_SIZE], axis=0, keepdims=True)
            for g in range(N_GROUPS)
        ],
        axis=0,
    )

    gids = []
    for _ in range(TOPK_GROUPS):
        gmval = jnp.max(gmax, axis=0, keepdims=True)
        gidx = jnp.min(
            jnp.where(gmax == gmval, giota, N_GROUPS), axis=0, keepdims=True
        )
        gids.append(gidx)
        gmax = jnp.where(giota == gidx, -jnp.inf, gmax)

    def ce(a, b):
        return jnp.minimum(a, b), jnp.maximum(a, b)

    g0, g1, g2, g3 = gids
    g0, g1 = ce(g0, g1)
    g2, g3 = ce(g2, g3)
    g0, g2 = ce(g0, g2)
    g1, g3 = ce(g1, g3)
    g1, g2 = ce(g1, g2)

    riota = jax.lax.broadcasted_iota(jnp.int32, (GROUP_SIZE, SUB), 0)
    crows, cidrows = [], []
    for gk in (g0, g1, g2, g3):
        c = scores[0:GROUP_SIZE]
        for g in range(1, N_GROUPS):
            c = jnp.where(gk == g, scores[g * GROUP_SIZE : (g + 1) * GROUP_SIZE], c)
        crows.append(c)
        cidrows.append(gk * GROUP_SIZE + riota)
    cand = jnp.concatenate(crows, axis=0)
    cidx = jnp.concatenate(cidrows, axis=0)

    wrows, irows = [], []
    for _ in range(TOPK):
        mval = jnp.max(cand, axis=0, keepdims=True)
        idx = jnp.min(
            jnp.where(cand == mval, cidx, NUM_EXPERTS), axis=0, keepdims=True
        )
        wrows.append(mval)
        irows.append(idx)
        cand = jnp.where(cidx == idx, -jnp.inf, cand)
    wvals = jnp.exp(jnp.concatenate(wrows, axis=0)) / denom
    ivals = jnp.concatenate(irows, axis=0)
    wout_ref[h * SUB : (h + 1) * SUB, :] = wvals.T
    iout_ref[h * SUB : (h + 1) * SUB, :] = ivals.T


def _gate_kernel(x_ref, w_ref, wout_ref, iout_ref):
    _gate_half(x_ref[...], w_ref[...], wout_ref, iout_ref, 0)


@jax.jit
def kernel(x, weight):
    T = x.shape[0]
    wout, iout = pl.pallas_call(
        _gate_kernel,
        grid=(T // TILE,),
        in_specs=[
            pl.BlockSpec((TILE, D_MODEL), lambda i: (i, 0)),
            pl.BlockSpec((NUM_EXPERTS, D_MODEL), lambda i: (0, 0)),
        ],
        out_specs=[
            pl.BlockSpec((TILE, TOPK), lambda i: (i, 0)),
            pl.BlockSpec((TILE, TOPK), lambda i: (i, 0)),
        ],
        out_shape=[
            jax.ShapeDtypeStruct((T, TOPK), jnp.float32),
            jax.ShapeDtypeStruct((T, TOPK), jnp.int32),
        ],
        compiler_params=pltpu.CompilerParams(
            dimension_semantics=("arbitrary",),
        ),
    )(x, weight)
    return wout, iout


# speedup vs baseline: 1.5981x; 1.5981x over previous
import jax
import jax.numpy as jnp
from jax.experimental import pallas as pl
from jax.experimental.pallas import tpu as pltpu

D_MODEL = 1024
NUM_EXPERTS = 64
TOPK = 8
N_GROUPS = 8
TOPK_GROUPS = 4
GROUP_SIZE = NUM_EXPERTS // N_GROUPS
N_CAND = TOPK_GROUPS * GROUP_SIZE

TILE = 4096
SPLIT = 1
SUB = TILE // SPLIT


def _gate_half(x, w, wout_ref, iout_ref, h):
    scores = jax.lax.dot_general(
        w, x, (((1,), (1,)), ((), ())), preferred_element_type=jnp.float32
    )
    denom = jnp.sum(jnp.exp(scores), axis=0, keepdims=True)

    giota = jax.lax.broadcasted_iota(jnp.int32, (N_GROUPS, SUB), 0)

    gmax = jnp.concatenate(
        [
            jnp.max(scores[g * GROUP_SIZE : (g + 1) * GROUP_SIZE], axis=0, keepdims=True)
            for g in range(N_GROUPS)
        ],
        axis=0,
    )

    gids = []
    for _ in range(TOPK_GROUPS):
        gmval = jnp.max(gmax, axis=0, keepdims=True)
        gidx = jnp.min(
            jnp.where(gmax == gmval, giota, N_GROUPS), axis=0, keepdims=True
        )
        gids.append(gidx)
        gmax = jnp.where(giota == gidx, -jnp.inf, gmax)

    def ce(a, b):
        return jnp.minimum(a, b), jnp.maximum(a, b)

    g0, g1, g2, g3 = gids
    g0, g1 = ce(g0, g1)
    g2, g3 = ce(g2, g3)
    g0, g2 = ce(g0, g2)
    g1, g3 = ce(g1, g3)
    g1, g2 = ce(g1, g2)

    riota = jax.lax.broadcasted_iota(jnp.int32, (GROUP_SIZE, SUB), 0)
    crows, cidrows = [], []
    for gk in (g0, g1, g2, g3):
        c = scores[0:GROUP_SIZE]
        for g in range(1, N_GROUPS):
            c = jnp.where(gk == g, scores[g * GROUP_SIZE : (g + 1) * GROUP_SIZE], c)
        crows.append(c)
        cidrows.append(gk * GROUP_SIZE + riota)
    cand = jnp.concatenate(crows, axis=0)
    cidx = jnp.concatenate(cidrows, axis=0)

    wrows, irows = [], []
    for _ in range(TOPK):
        mval = jnp.max(cand, axis=0, keepdims=True)
        idx = jnp.min(
            jnp.where(cand == mval, cidx, NUM_EXPERTS), axis=0, keepdims=True
        )
        wrows.append(mval)
        irows.append(idx)
        cand = jnp.where(cidx == idx, -jnp.inf, cand)
    wout_ref[:, h * SUB : (h + 1) * SUB] = jnp.exp(jnp.concatenate(wrows, axis=0)) / denom
    iout_ref[:, h * SUB : (h + 1) * SUB] = jnp.concatenate(irows, axis=0)


def _gate_kernel(x_ref, w_ref, wout_ref, iout_ref):
    _gate_half(x_ref[...], w_ref[...], wout_ref, iout_ref, 0)


@jax.jit
def kernel(x, weight):
    T = x.shape[0]
    wout, iout = pl.pallas_call(
        _gate_kernel,
        grid=(T // TILE,),
        in_specs=[
            pl.BlockSpec((TILE, D_MODEL), lambda i: (i, 0)),
            pl.BlockSpec((NUM_EXPERTS, D_MODEL), lambda i: (0, 0)),
        ],
        out_specs=[
            pl.BlockSpec((TOPK, TILE), lambda i: (0, i)),
            pl.BlockSpec((TOPK, TILE), lambda i: (0, i)),
        ],
        out_shape=[
            jax.ShapeDtypeStruct((TOPK, T), jnp.float32),
            jax.ShapeDtypeStruct((TOPK, T), jnp.int32),
        ],
        compiler_params=pltpu.CompilerParams(
            dimension_semantics=("arbitrary",),
        ),
    )(x, weight)
    return wout.T, iout.T
